# baseline (speedup 1.0000x reference)
import functools

import jax
import jax.numpy as jnp
from jax import lax
from jax.experimental import pallas as pl
from jax.experimental.pallas import tpu as pltpu
from jax.experimental.pallas import tpu_sc as plsc

N = 5000
NP = 5120
B = 1024
NB = NP // B
NBPAD = -(-NB // 8) * 8
CONF_THRE = 0.1
NMS_THRE = 0.45
_BITS_01 = 0x3DCCCCCD


def _sc_perm_kernel(direction):
    info = plsc.get_sparse_core_info()
    nw = info.num_cores * info.num_subcores
    bpw = NP // nw
    mesh = plsc.VectorSubcoreMesh(core_axis_name="c", subcore_axis_name="s")

    def body(rows_hbm, idx_hbm, out_hbm, idx_v, rows_v, sem):
        wid = lax.axis_index("s") * info.num_cores + lax.axis_index("c")
        base = wid * bpw
        pltpu.sync_copy(idx_hbm.at[pl.ds(base, bpw)], idx_v)
        if direction == "gather":
            pltpu.async_copy(rows_hbm.at[idx_v], rows_v, sem).wait()
            pltpu.sync_copy(rows_v, out_hbm.at[pl.ds(base, bpw)])
        else:
            pltpu.sync_copy(rows_hbm.at[pl.ds(base, bpw)], rows_v)
            pltpu.async_copy(rows_v, out_hbm.at[idx_v], sem).wait()

    return pl.kernel(
        body,
        mesh=mesh,
        out_type=jax.ShapeDtypeStruct((NP, 8), jnp.float32),
        scratch_types=[
            pltpu.VMEM((bpw,), jnp.int32),
            pltpu.VMEM((bpw, 8), jnp.float32),
            pltpu.SemaphoreType.DMA,
        ],
        compiler_params=pltpu.CompilerParams(use_tc_tiling_on_sc=False),
    )


def _sc_gather(table, idx):
    return _sc_perm_kernel("gather")(table, idx)


def _sc_scatter(rows, idx):
    return _sc_perm_kernel("scatter")(rows, idx)


def _dotrow(v, m):
    return jax.lax.dot_general(
        v, m, (((1,), (0,)), ((), ())), preferred_element_type=jnp.float32)


def _nms_body(cls_mm_ref, tri_ref, ts_ref, tt_ref, vblk_ref, dead_ref, sdets_ref):
    t = pl.program_id(0)
    kr = tri_ref[0, t]
    kc = tri_ref[1, t]

    @pl.when(t == 0)
    def _init():
        dead_ref[...] = 1.0 - vblk_ref[...]

    def mk_m():
        rx1 = ts_ref[:, 0:1]
        ry1 = ts_ref[:, 1:2]
        rx2 = ts_ref[:, 2:3]
        ry2 = ts_ref[:, 3:4]
        rcl = ts_ref[:, 5:6]
        cx1 = tt_ref[0:1, :]
        cy1 = tt_ref[1:2, :]
        cx2 = tt_ref[2:3, :]
        cy2 = tt_ref[3:4, :]
        ccl = tt_ref[5:6, :]
        w = jnp.maximum(jnp.minimum(rx2, cx2) - jnp.maximum(rx1, cx1), 0.0)
        h = jnp.maximum(jnp.minimum(ry2, cy2) - jnp.maximum(ry1, cy1), 0.0)
        inter = w * h
        ra = (rx2 - rx1) * (ry2 - ry1)
        ca = (cx2 - cx1) * (cy2 - cy1)
        union = ra + ca - inter
        return jnp.where((inter > NMS_THRE * union) & (rcl == ccl), 1.0, 0.0)

    act = (cls_mm_ref[1, kr] >= cls_mm_ref[0, kc]) & (
        cls_mm_ref[0, kr] <= cls_mm_ref[1, kc])

    @pl.when(kc == kr)
    def _intra():
        m = mk_m()
        sub = jax.lax.broadcasted_iota(jnp.int32, (B, B), 0)
        lane = jax.lax.broadcasted_iota(jnp.int32, (B, B), 1)
        mm = jnp.where(lane > sub, m, 0.0)
        dead0 = dead_ref[pl.ds(kr, 1), :]

        def cond(c):
            dd, da = c
            return jnp.sum((1.0 - dd) * (1.0 - da)) > 0.0

        def body(c):
            dd, da = c
            pot = _dotrow(1.0 - dd, mm)
            killed = _dotrow(da, mm)
            dd2 = jnp.maximum(dd, jnp.where(killed > 0.0, 1.0, 0.0))
            da2 = jnp.maximum(
                da, jnp.where((pot == 0.0) & (dd2 == 0.0), 1.0, 0.0))
            return (dd2, da2)

        dd, da = jax.lax.while_loop(cond, body, (dead0, jnp.zeros_like(dead0)))
        dead_ref[pl.ds(kr, 1), :] = dd
        eye = jnp.where(lane == sub, 1.0, 0.0)
        aliveT = jnp.sum(eye * da, axis=1, keepdims=True)
        sdets_ref[...] = ts_ref[...] * aliveT

    @pl.when((kc > kr) & act)
    def _cross():
        m = mk_m()
        alive = 1.0 - dead_ref[pl.ds(kr, 1), :]
        contrib = _dotrow(alive, m)
        cur = dead_ref[pl.ds(kc, 1), :]
        dead_ref[pl.ds(kc, 1), :] = jnp.maximum(
            cur, jnp.where(contrib > 0.0, 1.0, 0.0))


_NT = NB * (NB + 1) // 2
_TRI = [(r, c) for r in range(NB) for c in range(r, NB)]


def _nms_dead(cls_mm, table_sorted, tt, vblk, interpret=False):
    tri = jnp.array(
        [[r for r, _ in _TRI], [c for _, c in _TRI]], dtype=jnp.int32)
    grid_spec = pltpu.PrefetchScalarGridSpec(
        num_scalar_prefetch=2,
        grid=(_NT,),
        in_specs=[
            pl.BlockSpec((B, 8), lambda t, s, tr: (tr[0, t], 0)),
            pl.BlockSpec((8, B), lambda t, s, tr: (0, tr[1, t])),
            pl.BlockSpec((NBPAD, B), lambda t, s, tr: (0, 0)),
        ],
        out_specs=[
            pl.BlockSpec((NBPAD, B), lambda t, s, tr: (0, 0)),
            pl.BlockSpec((B, 8), lambda t, s, tr: (tr[0, t], 0)),
        ],
        scratch_shapes=[],
    )
    return pl.pallas_call(
        _nms_body,
        grid_spec=grid_spec,
        out_shape=[
            jax.ShapeDtypeStruct((NBPAD, B), jnp.float32),
            jax.ShapeDtypeStruct((NP, 8), jnp.float32),
        ],
        compiler_params=pltpu.CompilerParams(
            dimension_semantics=("arbitrary",),
        ),
        interpret=interpret,
    )(cls_mm, tri, table_sorted, tt, vblk)


def kernel(boxes, obj_conf, class_conf, class_ids):
    scores = obj_conf * class_conf
    valid = scores >= CONF_THRE
    sbits = jax.lax.bitcast_convert_type(scores, jnp.int32)
    skey = jnp.where(valid, sbits - (_BITS_01 - 1), 0)
    key = class_ids * (1 << 25) + ((1 << 25) - skey)
    order = jnp.argsort(key).astype(jnp.int32)
    ordp = jnp.concatenate([order, jnp.arange(N, NP, dtype=jnp.int32)])

    table = jnp.zeros((NP, 8), jnp.float32)
    feat = jnp.concatenate(
        [
            boxes,
            scores[:, None],
            class_ids.astype(jnp.float32)[:, None],
            valid.astype(jnp.float32)[:, None],
            jnp.ones((N, 1), jnp.float32),
        ],
        axis=1,
    )
    table = table.at[:N, :8].set(feat)

    ts = _sc_gather(table, ordp)
    tt = ts.T
    vs = ts[:, 6]
    vblk = jnp.zeros((NBPAD, B), jnp.float32).at[:NB, :].set(vs.reshape(NB, B))

    real = ts[:, 7] > 0.0
    cls_i = ts[:, 5].astype(jnp.int32)
    cls_lo = jnp.where(real, cls_i, 10**6).reshape(NB, B)
    cls_hi = jnp.where(real, cls_i, -1).reshape(NB, B)
    cls_mm = jnp.stack([jnp.min(cls_lo, axis=1), jnp.max(cls_hi, axis=1)])

    _, sdets = _nms_dead(cls_mm, ts, tt, vblk)
    out = _sc_scatter(sdets, ordp)
    return out[:N, :6]


# speedup vs baseline: 1.0237x; 1.0237x over previous
import functools

import jax
import jax.numpy as jnp
from jax import lax
from jax.experimental import pallas as pl
from jax.experimental.pallas import tpu as pltpu
from jax.experimental.pallas import tpu_sc as plsc

N = 5000
NP = 5120
B = 1024
NB = NP // B
NBPAD = -(-NB // 8) * 8
CONF_THRE = 0.1
NMS_THRE = 0.45
_BITS_01 = 0x3DCCCCCD


def _sc_perm_kernel(direction):
    info = plsc.get_sparse_core_info()
    nw = info.num_cores * info.num_subcores
    bpw = NP // nw
    mesh = plsc.VectorSubcoreMesh(core_axis_name="c", subcore_axis_name="s")

    def body(rows_hbm, idx_hbm, out_hbm, idx_v, rows_v, sem):
        wid = lax.axis_index("s") * info.num_cores + lax.axis_index("c")
        base = wid * bpw
        pltpu.sync_copy(idx_hbm.at[pl.ds(base, bpw)], idx_v)
        if direction == "gather":
            pltpu.async_copy(rows_hbm.at[idx_v], rows_v, sem).wait()
            pltpu.sync_copy(rows_v, out_hbm.at[pl.ds(base, bpw)])
        else:
            pltpu.sync_copy(rows_hbm.at[pl.ds(base, bpw)], rows_v)
            pltpu.async_copy(rows_v, out_hbm.at[idx_v], sem).wait()

    return pl.kernel(
        body,
        mesh=mesh,
        out_type=jax.ShapeDtypeStruct((NP, 16), jnp.float32),
        scratch_types=[
            pltpu.VMEM((bpw,), jnp.int32),
            pltpu.VMEM((bpw, 16), jnp.float32),
            pltpu.SemaphoreType.DMA,
        ],
        compiler_params=pltpu.CompilerParams(use_tc_tiling_on_sc=False),
    )


def _sc_gather(table, idx):
    return _sc_perm_kernel("gather")(table, idx)


def _sc_scatter(rows, idx):
    return _sc_perm_kernel("scatter")(rows, idx)


def _dotrow(v, m):
    return jax.lax.dot_general(
        v, m, (((1,), (0,)), ((), ())), preferred_element_type=jnp.float32)


def _nms_body(cls_mm_ref, tri_ref, ts_ref, tt_ref, vblk_ref, dead_ref, sdets_ref):
    t = pl.program_id(0)
    kr = tri_ref[0, t]
    kc = tri_ref[1, t]

    @pl.when(t == 0)
    def _init():
        dead_ref[...] = 1.0 - vblk_ref[...]

    def mk_m():
        rx1 = ts_ref[:, 0:1]
        ry1 = ts_ref[:, 1:2]
        rx2 = ts_ref[:, 2:3]
        ry2 = ts_ref[:, 3:4]
        rcl = ts_ref[:, 5:6]
        cx1 = tt_ref[0:1, :]
        cy1 = tt_ref[1:2, :]
        cx2 = tt_ref[2:3, :]
        cy2 = tt_ref[3:4, :]
        ccl = tt_ref[5:6, :]
        w = jnp.maximum(jnp.minimum(rx2, cx2) - jnp.maximum(rx1, cx1), 0.0)
        h = jnp.maximum(jnp.minimum(ry2, cy2) - jnp.maximum(ry1, cy1), 0.0)
        inter = w * h
        ra = (rx2 - rx1) * (ry2 - ry1)
        ca = (cx2 - cx1) * (cy2 - cy1)
        union = ra + ca - inter
        return jnp.where((inter > NMS_THRE * union) & (rcl == ccl), 1.0, 0.0)

    act = (cls_mm_ref[1, kr] >= cls_mm_ref[0, kc]) & (
        cls_mm_ref[0, kr] <= cls_mm_ref[1, kc])

    @pl.when(kc == kr)
    def _intra():
        m = mk_m()
        sub = jax.lax.broadcasted_iota(jnp.int32, (B, B), 0)
        lane = jax.lax.broadcasted_iota(jnp.int32, (B, B), 1)
        mm = jnp.where(lane > sub, m, 0.0)
        dead0 = dead_ref[pl.ds(kr, 1), :]

        def cond(c):
            dd, da = c
            return jnp.sum((1.0 - dd) * (1.0 - da)) > 0.0

        def body(c):
            dd, da = c
            pot = _dotrow(1.0 - dd, mm)
            killed = _dotrow(da, mm)
            dd2 = jnp.maximum(dd, jnp.where(killed > 0.0, 1.0, 0.0))
            da2 = jnp.maximum(
                da, jnp.where((pot == 0.0) & (dd2 == 0.0), 1.0, 0.0))
            return (dd2, da2)

        dd, da = jax.lax.while_loop(cond, body, (dead0, jnp.zeros_like(dead0)))
        dead_ref[pl.ds(kr, 1), :] = dd
        eye = jnp.where(lane == sub, 1.0, 0.0)
        aliveT = jnp.sum(eye * da, axis=1, keepdims=True)
        sdets_ref[...] = ts_ref[...] * aliveT

    @pl.when((kc > kr) & act)
    def _cross():
        m = mk_m()
        alive = 1.0 - dead_ref[pl.ds(kr, 1), :]
        contrib = _dotrow(alive, m)
        cur = dead_ref[pl.ds(kc, 1), :]
        dead_ref[pl.ds(kc, 1), :] = jnp.maximum(
            cur, jnp.where(contrib > 0.0, 1.0, 0.0))


_NT = NB * (NB + 1) // 2
_TRI = [(r, c) for r in range(NB) for c in range(r, NB)]


def _nms_dead(cls_mm, table_sorted, tt, vblk, interpret=False):
    tri = jnp.array(
        [[r for r, _ in _TRI], [c for _, c in _TRI]], dtype=jnp.int32)
    grid_spec = pltpu.PrefetchScalarGridSpec(
        num_scalar_prefetch=2,
        grid=(_NT,),
        in_specs=[
            pl.BlockSpec((B, 16), lambda t, s, tr: (tr[0, t], 0)),
            pl.BlockSpec((16, B), lambda t, s, tr: (0, tr[1, t])),
            pl.BlockSpec((NBPAD, B), lambda t, s, tr: (0, 0)),
        ],
        out_specs=[
            pl.BlockSpec((NBPAD, B), lambda t, s, tr: (0, 0)),
            pl.BlockSpec((B, 16), lambda t, s, tr: (tr[0, t], 0)),
        ],
        scratch_shapes=[],
    )
    return pl.pallas_call(
        _nms_body,
        grid_spec=grid_spec,
        out_shape=[
            jax.ShapeDtypeStruct((NBPAD, B), jnp.float32),
            jax.ShapeDtypeStruct((NP, 16), jnp.float32),
        ],
        compiler_params=pltpu.CompilerParams(
            dimension_semantics=("arbitrary",),
        ),
        interpret=interpret,
    )(cls_mm, tri, table_sorted, tt, vblk)


def kernel(boxes, obj_conf, class_conf, class_ids):
    scores = obj_conf * class_conf
    valid = scores >= CONF_THRE
    sbits = jax.lax.bitcast_convert_type(scores, jnp.int32)
    skey = jnp.where(valid, sbits - (_BITS_01 - 1), 0)
    key = class_ids * (1 << 25) + ((1 << 25) - skey)
    order = jnp.argsort(key).astype(jnp.int32)
    ordp = jnp.concatenate([order, jnp.arange(N, NP, dtype=jnp.int32)])

    table = jnp.zeros((NP, 16), jnp.float32)
    feat = jnp.concatenate(
        [
            boxes,
            scores[:, None],
            class_ids.astype(jnp.float32)[:, None],
            valid.astype(jnp.float32)[:, None],
            jnp.ones((N, 1), jnp.float32),
        ],
        axis=1,
    )
    table = table.at[:N, :8].set(feat)

    ts = _sc_gather(table, ordp)
    tt = ts.T
    vs = ts[:, 6]
    vblk = jnp.zeros((NBPAD, B), jnp.float32).at[:NB, :].set(vs.reshape(NB, B))

    real = ts[:, 7] > 0.0
    cls_i = ts[:, 5].astype(jnp.int32)
    cls_lo = jnp.where(real, cls_i, 10**6).reshape(NB, B)
    cls_hi = jnp.where(real, cls_i, -1).reshape(NB, B)
    cls_mm = jnp.stack([jnp.min(cls_lo, axis=1), jnp.max(cls_hi, axis=1)])

    _, sdets = _nms_dead(cls_mm, ts, tt, vblk)
    out = _sc_scatter(sdets, ordp)
    return out[:N, :6]
